# baseline (speedup 1.0000x reference)
import functools

import jax
import jax.numpy as jnp
from jax import lax
from jax.experimental import pallas as pl
from jax.experimental.pallas import tpu as pltpu
from jax.experimental.pallas import tpu_sc as plsc

_VOCAB = 1000000
_EMBED = 32
_LABELS = 107
_ORDER = 3
_HASHES = 20
_BATCH = 16384
_PER_ROW = _ORDER * _HASHES

_NC = 2
_NS = 16
_NW = _NC * _NS
_SLOT = 64
_GS = 2 * _SLOT
_VROWS = _BATCH // 2
_VPW = _VROWS // _NW
_NG = 8
_NCHUNK = _VPW // _NG


_FB = 1024


def _pack64(x, zero):
    return jnp.concatenate(
        [x[:, 0, :], x[:, 1, :], x[:, 2, :], zero], axis=-1)


def _flat_body(lo_i, hi_i, lo_w, hi_w, oi_ref, ow_ref):
    pid = pl.program_id(0)
    zi = (lax.broadcasted_iota(jnp.int32, (_FB, _SLOT - _PER_ROW), 0)
          + pid * _FB) * 29 + lax.broadcasted_iota(
              jnp.int32, (_FB, _SLOT - _PER_ROW), 1)
    zw = jnp.zeros((_FB, _SLOT - _PER_ROW), jnp.float32)
    oi_ref[...] = jnp.concatenate(
        [_pack64(lo_i[...], zi), _pack64(hi_i[...], zi)], axis=-1)
    ow_ref[...] = jnp.concatenate(
        [_pack64(lo_w[...], zw), _pack64(hi_w[...], zw)], axis=-1)


def _flatten(ngrams, weights):
    nblk = _VROWS // _FB
    spec_lo = pl.BlockSpec((_FB, _ORDER, _HASHES), lambda i: (i, 0, 0))
    spec_hi = pl.BlockSpec((_FB, _ORDER, _HASHES), lambda i: (i + nblk, 0, 0))
    return pl.pallas_call(
        _flat_body,
        grid=(nblk,),
        in_specs=[spec_lo, spec_hi, spec_lo, spec_hi],
        out_specs=[
            pl.BlockSpec((_FB, _GS), lambda i: (i, 0)),
            pl.BlockSpec((_FB, _GS), lambda i: (i, 0)),
        ],
        out_shape=[
            jax.ShapeDtypeStruct((_VROWS, _GS), jnp.int32),
            jax.ShapeDtypeStruct((_VROWS, _GS), jnp.float32),
        ],
    )(ngrams, ngrams, weights, weights)


def _sc_body(idx_hbm, w_hbm, emb_hbm, out_hbm, idx_v, w_v, rows_v, out_v,
             sems):
    wid = lax.axis_index("s") * _NC + lax.axis_index("c")
    vbase = wid * _VPW
    _CW = _NG * _GS

    def stage_and_fire(g, buf):
        vr0 = vbase + g * _NG
        pltpu.sync_copy(idx_hbm.at[pl.ds(vr0, _NG)], idx_v.at[buf])
        pltpu.sync_copy(w_hbm.at[pl.ds(vr0, _NG)], w_v.at[buf])
        for j in range(_NG):
            pltpu.async_copy(
                emb_hbm.at[idx_v.at[buf, j]],
                rows_v.at[buf, pl.ds(j * _GS, _GS)],
                sems.at[buf],
            )

    def drain(buf):
        pltpu.make_async_copy(
            emb_hbm.at[pl.ds(0, _NG * _GS)],
            rows_v.at[buf],
            sems.at[buf],
        ).wait()

    def compute(g, buf):
        vr0 = vbase + g * _NG

        def bb_body(bb, carry):
            for q in range(2):
                for o in range(_ORDER):
                    acc0 = jnp.zeros((16,), jnp.float32)
                    acc1 = jnp.zeros((16,), jnp.float32)
                    col0 = q * _SLOT + o * _HASHES
                    wv0 = w_v[buf, pl.ds(bb * _GS + col0, 16)]
                    wv1 = w_v[buf, pl.ds(bb * _GS + col0 + 4, 16)]
                    for h in range(_HASHES):
                        w = wv0[h] if h < 16 else wv1[h - 4]
                        wb = jnp.full((16,), w, jnp.float32)
                        r = bb * _GS + col0 + h
                        acc0 = acc0 + wb * rows_v[buf, r, pl.ds(0, 16)]
                        acc1 = acc1 + wb * rows_v[buf, r, pl.ds(16, 16)]
                    out_v[q, bb, pl.ds(o * _EMBED, 16)] = acc0
                    out_v[q, bb, pl.ds(o * _EMBED + 16, 16)] = acc1
            return carry

        lax.fori_loop(0, _NG, bb_body, 0)
        pltpu.sync_copy(out_v.at[0], out_hbm.at[pl.ds(vr0, _NG)])
        pltpu.sync_copy(out_v.at[1], out_hbm.at[pl.ds(_VROWS + vr0, _NG)])

    def loop_body(g, carry):
        vr0 = vbase + g * _NG
        flat0 = vr0 * _GS
        pltpu.sync_copy(idx_hbm.at[pl.ds(flat0, _CW)], idx_v.at[0])
        pltpu.sync_copy(w_hbm.at[pl.ds(flat0, _CW)], w_v.at[0])
        copies = [
            pltpu.async_copy(
                emb_hbm.at[idx_v.at[0, pl.ds(j * _GS, _GS)]],
                rows_v.at[0, pl.ds(j * _GS, _GS)],
                sems.at[0],
            )
            for j in range(_NG)
        ]
        for c in copies:
            c.wait()
        compute(g, 0)
        return carry

    lax.fori_loop(0, _NCHUNK, loop_body, 0)


def _sc_gather(idx2d, w2d, emb):
    mesh = plsc.VectorSubcoreMesh(core_axis_name="c", subcore_axis_name="s")
    k = functools.partial(
        pl.kernel,
        mesh=mesh,
        compiler_params=pltpu.CompilerParams(use_tc_tiling_on_sc=False),
        out_type=jax.ShapeDtypeStruct((_BATCH, _ORDER * _EMBED), jnp.float32),
        scratch_types=[
            pltpu.VMEM((2, _NG * _GS), jnp.int32),
            pltpu.VMEM((2, _NG * _GS), jnp.float32),
            pltpu.VMEM((2, _NG * _GS, _EMBED), jnp.float32),
            pltpu.VMEM((2, _NG, _ORDER * _EMBED), jnp.float32),
            pltpu.SemaphoreType.DMA((2,)),
        ],
    )(_sc_body)
    return k(idx2d, w2d, emb)


_MLP_BLK = 1024


def _mlp_body(e_ref, w1_ref, b1_ref, w2_ref, b2_ref, o_ref):
    e = e_ref[...]
    h = lax.dot_general(e, w1_ref[...], (((1,), (1,)), ((), ())),
                        preferred_element_type=jnp.float32) + b1_ref[...]
    l = lax.dot_general(h, w2_ref[...], (((1,), (1,)), ((), ())),
                        preferred_element_type=jnp.float32) + b2_ref[...]
    m = jnp.max(l, axis=-1, keepdims=True)
    lse = jnp.log(jnp.sum(jnp.exp(l - m), axis=-1, keepdims=True)) + m
    o_ref[...] = l - lse


def _mlp(embed, W1, b1, W2, b2):
    grid = (_BATCH // _MLP_BLK,)
    return pl.pallas_call(
        _mlp_body,
        grid=grid,
        in_specs=[
            pl.BlockSpec((_MLP_BLK, _ORDER * _EMBED), lambda i: (i, 0)),
            pl.BlockSpec((_EMBED, _ORDER * _EMBED), lambda i: (0, 0)),
            pl.BlockSpec((1, _EMBED), lambda i: (0, 0)),
            pl.BlockSpec((_LABELS, _EMBED), lambda i: (0, 0)),
            pl.BlockSpec((1, _LABELS), lambda i: (0, 0)),
        ],
        out_specs=pl.BlockSpec((_MLP_BLK, _LABELS), lambda i: (i, 0)),
        out_shape=jax.ShapeDtypeStruct((_BATCH, _LABELS), jnp.float32),
    )(embed, W1, b1, W2, b2)


def kernel(ngrams, ngrams_weights, emb, W1, b1, W2, b2):
    idx2d, w2d = _flatten(ngrams, ngrams_weights)
    one = lax.optimization_barrier(jnp.float32(1.0))
    embed = _sc_gather(idx2d.reshape(-1), w2d.reshape(-1), emb * one)
    return _mlp(embed, W1, b1.reshape(1, -1), W2, b2.reshape(1, -1))


# speedup vs baseline: 1.2018x; 1.2018x over previous
import functools

import jax
import jax.numpy as jnp
from jax import lax
from jax.experimental import pallas as pl
from jax.experimental.pallas import tpu as pltpu
from jax.experimental.pallas import tpu_sc as plsc

_VOCAB = 1000000
_EMBED = 32
_LABELS = 107
_ORDER = 3
_HASHES = 20
_BATCH = 16384
_PER_ROW = _ORDER * _HASHES

_NC = 2
_NS = 16
_NW = _NC * _NS
_SLOT = 64
_GS = 2 * _SLOT
_VROWS = _BATCH // 2
_VPW = _VROWS // _NW
_NG = 8
_NCHUNK = _VPW // _NG


_FB = 1024


def _pack64(x, zero):
    return jnp.concatenate(
        [x[:, 0, :], x[:, 1, :], x[:, 2, :], zero], axis=-1)


def _flat_body(lo_i, hi_i, lo_w, hi_w, oi_ref, ow_ref):
    pid = pl.program_id(0)
    zi = (lax.broadcasted_iota(jnp.int32, (_FB, _SLOT - _PER_ROW), 0)
          + pid * _FB) * 29 + lax.broadcasted_iota(
              jnp.int32, (_FB, _SLOT - _PER_ROW), 1)
    zw = jnp.zeros((_FB, _SLOT - _PER_ROW), jnp.float32)
    ti = (lo_i[...] % _NQ) * 4 + lo_i[...] // _NQ
    th = (hi_i[...] % _NQ) * 4 + hi_i[...] // _NQ
    oi_ref[...] = jnp.concatenate(
        [_pack64(ti, zi), _pack64(th, zi)], axis=-1)
    ow_ref[...] = jnp.concatenate(
        [_pack64(lo_w[...], zw), _pack64(hi_w[...], zw)], axis=-1)


def _flatten(ngrams, weights):
    nblk = _VROWS // _FB
    spec_lo = pl.BlockSpec((_FB, _ORDER, _HASHES), lambda i: (i, 0, 0))
    spec_hi = pl.BlockSpec((_FB, _ORDER, _HASHES), lambda i: (i + nblk, 0, 0))
    return pl.pallas_call(
        _flat_body,
        grid=(nblk,),
        in_specs=[spec_lo, spec_hi, spec_lo, spec_hi],
        out_specs=[
            pl.BlockSpec((_FB, _GS), lambda i: (i, 0)),
            pl.BlockSpec((_FB, _GS), lambda i: (i, 0)),
        ],
        out_shape=[
            jax.ShapeDtypeStruct((_VROWS, _GS), jnp.int32),
            jax.ShapeDtypeStruct((_VROWS, _GS), jnp.float32),
        ],
    )(ngrams, ngrams, weights, weights)


_NQ = _VOCAB // 4
_TBLK = 2000


def _packt_body(a, b, c, d, o_ref):
    o_ref[...] = jnp.concatenate([a[...], b[...], c[...], d[...]], axis=-1)


def _pack_table(emb):
    nb = _NQ // _TBLK
    mk = lambda k: pl.BlockSpec((_TBLK, _EMBED), lambda i, k=k: (i + k * nb, 0))
    return pl.pallas_call(
        _packt_body,
        grid=(nb,),
        in_specs=[mk(0), mk(1), mk(2), mk(3)],
        out_specs=pl.BlockSpec((_TBLK, 4 * _EMBED), lambda i: (i, 0)),
        out_shape=jax.ShapeDtypeStruct((_NQ, 4 * _EMBED), jnp.float32),
    )(emb, emb, emb, emb)


def _sc_body(idx_hbm, w_hbm, emb_hbm, out_hbm, idx_v, w_v, rows_v, out_v,
             sems):
    wid = lax.axis_index("s") * _NC + lax.axis_index("c")
    vbase = wid * _VPW
    _CW = _NG * _GS

    def stage_and_fire(g, buf):
        vr0 = vbase + g * _NG
        pltpu.sync_copy(idx_hbm.at[pl.ds(vr0, _NG)], idx_v.at[buf])
        pltpu.sync_copy(w_hbm.at[pl.ds(vr0, _NG)], w_v.at[buf])
        for j in range(_NG):
            pltpu.async_copy(
                emb_hbm.at[idx_v.at[buf, j]],
                rows_v.at[buf, pl.ds(j * _GS, _GS)],
                sems.at[buf],
            )

    def drain(buf):
        pltpu.make_async_copy(
            emb_hbm.at[pl.ds(0, _NG * _GS)],
            rows_v.at[buf],
            sems.at[buf],
        ).wait()

    def compute(g, buf):
        vr0 = vbase + g * _NG

        def bb_body(bb, carry):
            for q in range(2):
                for o in range(_ORDER):
                    acc0 = jnp.zeros((16,), jnp.float32)
                    acc1 = jnp.zeros((16,), jnp.float32)
                    col0 = q * _SLOT + o * _HASHES
                    wv0 = w_v[buf, pl.ds(bb * _GS + col0, 16)]
                    wv1 = w_v[buf, pl.ds(bb * _GS + col0 + 4, 16)]
                    for h in range(_HASHES):
                        w = wv0[h] if h < 16 else wv1[h - 4]
                        wb = jnp.full((16,), w, jnp.float32)
                        r = bb * _GS + col0 + h
                        acc0 = acc0 + wb * rows_v[buf, r, pl.ds(0, 16)]
                        acc1 = acc1 + wb * rows_v[buf, r, pl.ds(16, 16)]
                    out_v[q, bb, pl.ds(o * _EMBED, 16)] = acc0
                    out_v[q, bb, pl.ds(o * _EMBED + 16, 16)] = acc1
            return carry

        lax.fori_loop(0, _NG, bb_body, 0)
        pltpu.sync_copy(out_v.at[0], out_hbm.at[pl.ds(vr0, _NG)])
        pltpu.sync_copy(out_v.at[1], out_hbm.at[pl.ds(_VROWS + vr0, _NG)])

    def loop_body(g, carry):
        vr0 = vbase + g * _NG
        flat0 = vr0 * _GS
        pltpu.sync_copy(idx_hbm.at[pl.ds(flat0, _CW)], idx_v.at[0])
        pltpu.sync_copy(w_hbm.at[pl.ds(flat0, _CW)], w_v.at[0])
        copies = [
            pltpu.async_copy(
                emb_hbm.at[idx_v.at[0, pl.ds(j * _GS, _GS)]],
                rows_v.at[0, pl.ds(j * _GS, _GS)],
                sems.at[0],
            )
            for j in range(_NG)
        ]
        for c in copies:
            c.wait()
        compute(g, 0)
        return carry

    lax.fori_loop(0, _NCHUNK, loop_body, 0)


def _sc_gather(idx2d, w2d, emb):
    mesh = plsc.VectorSubcoreMesh(core_axis_name="c", subcore_axis_name="s")
    k = functools.partial(
        pl.kernel,
        mesh=mesh,
        compiler_params=pltpu.CompilerParams(use_tc_tiling_on_sc=False),
        out_type=jax.ShapeDtypeStruct((_BATCH, _ORDER * _EMBED), jnp.float32),
        scratch_types=[
            pltpu.VMEM((2, _NG * _GS), jnp.int32),
            pltpu.VMEM((2, _NG * _GS), jnp.float32),
            pltpu.VMEM((2, _NG * _GS, _EMBED), jnp.float32),
            pltpu.VMEM((2, _NG, _ORDER * _EMBED), jnp.float32),
            pltpu.SemaphoreType.DMA((2,)),
        ],
    )(_sc_body)
    return k(idx2d, w2d, emb)


_MLP_BLK = 1024


def _mlp_body(e_ref, w1_ref, b1_ref, w2_ref, b2_ref, o_ref):
    e = e_ref[...]
    h = lax.dot_general(e, w1_ref[...], (((1,), (1,)), ((), ())),
                        preferred_element_type=jnp.float32) + b1_ref[...]
    l = lax.dot_general(h, w2_ref[...], (((1,), (1,)), ((), ())),
                        preferred_element_type=jnp.float32) + b2_ref[...]
    m = jnp.max(l, axis=-1, keepdims=True)
    lse = jnp.log(jnp.sum(jnp.exp(l - m), axis=-1, keepdims=True)) + m
    o_ref[...] = l - lse


def _mlp(embed, W1, b1, W2, b2):
    grid = (_BATCH // _MLP_BLK,)
    return pl.pallas_call(
        _mlp_body,
        grid=grid,
        in_specs=[
            pl.BlockSpec((_MLP_BLK, _ORDER * _EMBED), lambda i: (i, 0)),
            pl.BlockSpec((_EMBED, _ORDER * _EMBED), lambda i: (0, 0)),
            pl.BlockSpec((1, _EMBED), lambda i: (0, 0)),
            pl.BlockSpec((_LABELS, _EMBED), lambda i: (0, 0)),
            pl.BlockSpec((1, _LABELS), lambda i: (0, 0)),
        ],
        out_specs=pl.BlockSpec((_MLP_BLK, _LABELS), lambda i: (i, 0)),
        out_shape=jax.ShapeDtypeStruct((_BATCH, _LABELS), jnp.float32),
    )(embed, W1, b1, W2, b2)


def kernel(ngrams, ngrams_weights, emb, W1, b1, W2, b2):
    idx2d, w2d = _flatten(ngrams, ngrams_weights)
    embp = _pack_table(emb).reshape(_VOCAB, _EMBED)
    embed = _sc_gather(idx2d.reshape(-1), w2d.reshape(-1), embp)
    return _mlp(embed, W1, b1.reshape(1, -1), W2, b2.reshape(1, -1))
